# baseline (speedup 1.0000x reference)
import functools

import jax
import jax.numpy as jnp
from jax import lax
from jax.experimental import pallas as pl
from jax.experimental.pallas import tpu as pltpu
from jax.experimental.pallas import tpu_sc as plsc

N_NODES = 10000
N_PAD = 10240
DEN_R = N_PAD // 16
D_HID = 128
D_HALF = 64
N_EDGES = 320000
E_TOT = N_EDGES + N_NODES
NC, NS = 2, 16
K_EDGES = 256
ROWS = K_EDGES // 128
NCHUNK = 84
E_PAD = NS * NCHUNK * K_EDGES
E_PAD2 = E_PAD + 2 * K_EDGES
ROWS_PER_TILE = NCHUNK * ROWS
SLICE = N_PAD // NS
DSLICE = DEN_R // NS
HR = SLICE // 128


def _dense_body(x_ref, W_ref, as_ref, ad_ref, h_ref, asrc_ref, adst_ref):
    h = jnp.dot(x_ref[...], W_ref[...], preferred_element_type=jnp.float32)
    h_ref[...] = h
    asrc_ref[...] = jnp.dot(h, as_ref[...], preferred_element_type=jnp.float32)
    adst_ref[...] = jnp.dot(h, ad_ref[...], preferred_element_type=jnp.float32)


def _head_body(num_ref, den_ref, bias_ref, W2_ref, b2_ref, y_ref):
    num = jnp.concatenate(
        [num_ref[0:N_PAD, :], num_ref[N_PAD:2 * N_PAD, :]], axis=1)
    den = den_ref[0:N_PAD, 0:1]
    out = num / (den + 1e-16) + bias_ref[...]
    out = jnp.maximum(out, 0.0)
    y = jnp.dot(out, W2_ref[...], preferred_element_type=jnp.float32) + b2_ref[...]
    y_ref[...] = jax.nn.sigmoid(y)


def _sc_edge_body(src_hbm, dst_hbm, asrc_hbm, adst_hbm, h2_hbm,
                  num_out, den_out,
                  src_v0, src_v1, dst_v0, dst_v1, dstsc_v,
                  as_e0, as_e1, ad_e0, ad_e1,
                  den_loc, ident_v, ident2_v, w_v,
                  rows_v0, rows_v1,
                  h_loc, asrc_sh, adst_sh, num_acc, den_acc,
                  sem_g0, sem_g1, sem_h0, sem_h1, sem_s, sem_t):
    c = lax.axis_index("c")
    s = lax.axis_index("s")
    src_v = (src_v0, src_v1)
    dst_v = (dst_v0, dst_v1)
    rows_v = (rows_v0, rows_v1)
    as_e = (as_e0, as_e1)
    ad_e = (ad_e0, ad_e1)
    sem_g = (sem_g0, sem_g1)
    sem_h = (sem_h0, sem_h1)

    def _scatter_descs(b):
        return [pltpu.make_async_copy(
                    rows_v[b].at[pl.ds(gr * 128, 128)],
                    num_acc.at[dstsc_v.at[gr]], sem_s)
                for gr in range(ROWS)]

    def _stage(g, b, sync):
        rb = s * ROWS_PER_TILE + g * ROWS
        if sync:
            pltpu.sync_copy(src_hbm.at[pl.ds(rb, ROWS)], src_v[b])
            pltpu.sync_copy(dst_hbm.at[pl.ds(rb, ROWS)], dst_v[b])
        else:
            pltpu.async_copy(src_hbm.at[pl.ds(rb, ROWS)], src_v[b], sem_t)
            pltpu.async_copy(dst_hbm.at[pl.ds(rb, ROWS)], dst_v[b], sem_t)

    def _stage_drain():
        pltpu.make_async_copy(src_hbm.at[pl.ds(0, ROWS)], src_v0, sem_t).wait()
        pltpu.make_async_copy(dst_hbm.at[pl.ds(0, ROWS)], dst_v0, sem_t).wait()

    def _fire_gathers(b):
        for gr in range(ROWS):
            pltpu.async_copy(asrc_sh.at[src_v[b].at[gr]],
                             as_e[b].at[pl.ds(gr * 128, 128)], sem_g[b])
            pltpu.async_copy(adst_sh.at[dst_v[b].at[gr]],
                             ad_e[b].at[pl.ds(gr * 128, 128)], sem_g[b])
            pltpu.async_copy(h_loc.at[src_v[b].at[gr]],
                             rows_v[b].at[pl.ds(gr * 128, 128)], sem_h[b])

    def _gather_descs(b):
        ds_ = []
        for gr in range(ROWS):
            ds_.append(pltpu.make_async_copy(
                asrc_sh.at[src_v[b].at[gr]],
                as_e[b].at[pl.ds(gr * 128, 128)], sem_g[b]))
            ds_.append(pltpu.make_async_copy(
                adst_sh.at[dst_v[b].at[gr]],
                ad_e[b].at[pl.ds(gr * 128, 128)], sem_g[b]))
        return ds_

    def _h_descs(b):
        return [pltpu.make_async_copy(
                    h_loc.at[src_v[b].at[gr]],
                    rows_v[b].at[pl.ds(gr * 128, 128)], sem_h[b])
                for gr in range(ROWS)]

    for gr in range(DEN_R // 128):
        for t in range(8):
            ident_v[gr, pl.ds(16 * t, 16)] = (
                jnp.full((16,), gr * 128 + t * 16, jnp.int32)
                + lax.iota(jnp.int32, 16))
    for gr in range(HR):
        for t in range(8):
            ident2_v[gr, pl.ds(16 * t, 16)] = (
                (jnp.full((16,), s * SLICE + gr * 128 + t * 16, jnp.int32)
                 + lax.iota(jnp.int32, 16)) * 2 + c)

    pltpu.sync_copy(asrc_hbm.at[pl.ds(s * SLICE, SLICE)],
                    asrc_sh.at[pl.ds(s * SLICE, SLICE)])
    pltpu.sync_copy(adst_hbm.at[pl.ds(s * SLICE, SLICE)],
                    adst_sh.at[pl.ds(s * SLICE, SLICE)])
    for gr in range(HR):
        pltpu.async_copy(h2_hbm.at[ident2_v.at[gr]],
                         rows_v0.at[pl.ds(0, 128)], sem_g0).wait()
        pltpu.sync_copy(rows_v0.at[pl.ds(0, 128)],
                        h_loc.at[pl.ds(s * SLICE + gr * 128, 128)])

    def _zero_body(r, carry):
        for d in range(4):
            rows_v0[r, pl.ds(16 * d, 16)] = jnp.zeros((16,), jnp.float32)
            rows_v1[r, pl.ds(16 * d, 16)] = jnp.zeros((16,), jnp.float32)
        return carry
    lax.fori_loop(0, K_EDGES, _zero_body, 0)

    def _zden_body(r, carry):
        den_loc[r, :] = jnp.zeros((16,), jnp.float32)
        return carry
    lax.fori_loop(0, DEN_R, _zden_body, 0)
    for gr in range(ROWS):
        for t in range(8):
            dstsc_v[gr, pl.ds(16 * t, 16)] = jnp.zeros((16,), jnp.int32)

    base = s * SLICE
    pltpu.sync_copy(rows_v0, num_acc.at[pl.ds(base, K_EDGES)])
    pltpu.sync_copy(rows_v1, num_acc.at[pl.ds(base + K_EDGES, K_EDGES)])
    pltpu.sync_copy(rows_v0.at[pl.ds(0, SLICE - 2 * K_EDGES)],
                    num_acc.at[pl.ds(base + 2 * K_EDGES,
                                     SLICE - 2 * K_EDGES)])
    pltpu.sync_copy(den_loc.at[pl.ds(0, DSLICE)],
                    den_acc.at[pl.ds(s * DSLICE, DSLICE)])
    plsc.subcore_barrier()

    for gr in range(ROWS):
        pltpu.async_copy(rows_v1.at[pl.ds(gr * 128, 128)],
                         num_acc.at[dstsc_v.at[gr]], sem_s, add=True)
    _stage(0, 0, sync=True)
    _stage(1, 1, sync=False)
    _fire_gathers(0)

    def _chunk(g, b):
        for d_ in _gather_descs(b):
            d_.wait()
        for d_ in _scatter_descs(1 - b):
            d_.wait()
        _stage_drain()
        _fire_gathers(1 - b)

        def _w_body(gg, carry2):
            for t in range(8):
                sl = pl.ds(16 * t, 16)
                sle = pl.ds(gg * 128 + t * 16, 16)
                di = dst_v[b][gg, sl]
                e = as_e[b][sle] + ad_e[b][sle]
                e = jnp.where(e < 0, e * jnp.float32(0.2), e)
                w = jnp.exp(e)
                w_v[sle] = w
                plsc.addupdate_scatter(den_loc, [di >> 4, di & 15], w)
            return carry2
        lax.fori_loop(0, ROWS, _w_body, 0)

        for d_ in _h_descs(b):
            d_.wait()

        def _scale_blk(bk, carry2):
            r0 = bk * 16
            w16 = w_v[pl.ds(r0, 16)]
            for l in range(16):
                wb = jnp.broadcast_to(w16[l], (16,))
                for d in range(4):
                    sl = pl.ds(16 * d, 16)
                    rows_v[b][r0 + l, sl] = rows_v[b][r0 + l, sl] * wb
            return carry2
        lax.fori_loop(0, K_EDGES // 16, _scale_blk, 0)

        for gr in range(ROWS):
            for t in range(8):
                sl = pl.ds(16 * t, 16)
                dstsc_v[gr, sl] = dst_v[b][gr, sl]
        for gr in range(ROWS):
            pltpu.async_copy(rows_v[b].at[pl.ds(gr * 128, 128)],
                             num_acc.at[dstsc_v.at[gr]], sem_s, add=True)

        _stage(g + 2, b, sync=False)

    def _outer(g2, carry):
        _chunk(g2 * 2, 0)
        _chunk(g2 * 2 + 1, 1)
        return carry
    lax.fori_loop(0, NCHUNK // 2, _outer, 0)

    for d_ in _gather_descs(0):
        d_.wait()
    for d_ in _h_descs(0):
        d_.wait()
    _stage_drain()
    for d_ in _scatter_descs(1):
        d_.wait()

    for gr in range(DEN_R // 128):
        pltpu.sync_copy(den_loc.at[pl.ds(gr * 128, 128)],
                        den_acc.at[ident_v.at[gr]], add=True)

    plsc.subcore_barrier()
    ob = c * N_PAD + s * SLICE
    pltpu.sync_copy(num_acc.at[pl.ds(s * SLICE, SLICE)],
                    num_out.at[pl.ds(ob, SLICE)])
    pltpu.sync_copy(den_acc.at[pl.ds(s * DSLICE, DSLICE)],
                    den_out.at[pl.ds(c * DEN_R + s * DSLICE, DSLICE)])


_sc_edge = functools.partial(
    pl.kernel,
    out_type=[
        jax.ShapeDtypeStruct((2 * N_PAD, D_HALF), jnp.float32),
        jax.ShapeDtypeStruct((2 * DEN_R, 16), jnp.float32),
    ],
    mesh=plsc.VectorSubcoreMesh(core_axis_name="c", subcore_axis_name="s"),
    compiler_params=pltpu.CompilerParams(needs_layout_passes=False,
                                         use_tc_tiling_on_sc=False),
    scratch_types=[
        pltpu.VMEM((ROWS, 128), jnp.int32),
        pltpu.VMEM((ROWS, 128), jnp.int32),
        pltpu.VMEM((ROWS, 128), jnp.int32),
        pltpu.VMEM((ROWS, 128), jnp.int32),
        pltpu.VMEM((ROWS, 128), jnp.int32),
        pltpu.VMEM((K_EDGES,), jnp.float32),
        pltpu.VMEM((K_EDGES,), jnp.float32),
        pltpu.VMEM((K_EDGES,), jnp.float32),
        pltpu.VMEM((K_EDGES,), jnp.float32),
        pltpu.VMEM((DEN_R, 16), jnp.float32),
        pltpu.VMEM((DEN_R // 128, 128), jnp.int32),
        pltpu.VMEM((HR, 128), jnp.int32),
        pltpu.VMEM((K_EDGES,), jnp.float32),
        pltpu.VMEM((K_EDGES, D_HALF), jnp.float32),
        pltpu.VMEM((K_EDGES, D_HALF), jnp.float32),
        pltpu.VMEM_SHARED((N_PAD, D_HALF), jnp.float32),
        pltpu.VMEM_SHARED((N_PAD,), jnp.float32),
        pltpu.VMEM_SHARED((N_PAD,), jnp.float32),
        pltpu.VMEM_SHARED((N_PAD, D_HALF), jnp.float32),
        pltpu.VMEM_SHARED((DEN_R, 16), jnp.float32),
        pltpu.SemaphoreType.DMA,
        pltpu.SemaphoreType.DMA,
        pltpu.SemaphoreType.DMA,
        pltpu.SemaphoreType.DMA,
        pltpu.SemaphoreType.DMA,
        pltpu.SemaphoreType.DMA,
    ],
)(_sc_edge_body)


def kernel(x, edge_index, W, att_src, att_dst, bias, W2, b2):
    h, a_src, a_dst = pl.pallas_call(
        _dense_body,
        out_shape=[
            jax.ShapeDtypeStruct((N_NODES, D_HID), jnp.float32),
            jax.ShapeDtypeStruct((N_NODES, 1), jnp.float32),
            jax.ShapeDtypeStruct((N_NODES, 1), jnp.float32),
        ],
    )(x, W, att_src.reshape(D_HID, 1), att_dst.reshape(D_HID, 1))

    loop = jnp.arange(N_NODES, dtype=jnp.int32)
    src = jnp.concatenate([edge_index[0].astype(jnp.int32), loop,
                           jnp.zeros((E_PAD2 - E_TOT,), jnp.int32)])
    dst = jnp.concatenate([edge_index[1].astype(jnp.int32), loop,
                           jnp.full((E_PAD2 - E_TOT,), N_NODES, jnp.int32)])
    src2 = src.reshape(E_PAD2 // 128, 128)
    dst2 = dst.reshape(E_PAD2 // 128, 128)
    asrc_p = jnp.pad(a_src[:, 0], (0, N_PAD - N_NODES))
    adst_p = jnp.pad(a_dst[:, 0], (0, N_PAD - N_NODES))
    hp = jnp.pad(h, ((0, N_PAD - N_NODES), (0, 0)))
    h2 = hp.reshape(2 * N_PAD, D_HALF)

    num_flat, den_flat = _sc_edge(src2, dst2, asrc_p, adst_p, h2)
    den = den_flat[:DEN_R].reshape(N_PAD, 1)

    y = pl.pallas_call(
        _head_body,
        out_shape=jax.ShapeDtypeStruct((N_PAD, 1), jnp.float32),
    )(num_flat, den, bias.reshape(1, D_HID), W2, b2.reshape(1, 1))
    return y[:N_NODES]


# speedup vs baseline: 1.3220x; 1.3220x over previous
import functools

import jax
import jax.numpy as jnp
from jax import lax
from jax.experimental import pallas as pl
from jax.experimental.pallas import tpu as pltpu
from jax.experimental.pallas import tpu_sc as plsc

N_NODES = 10000
N_PAD = 10240
DEN_HALF = N_PAD // 2
DEN_RR = DEN_HALF + 128
DEN_SL = DEN_RR // 16
D_HID = 128
D_HALF = 64
N_EDGES = 320000
E_TOT = N_EDGES + N_NODES
NC, NS = 2, 16
K_EDGES = 256
ROWS = K_EDGES // 128
NCHUNK = 84
E_PAD = NS * NCHUNK * K_EDGES
E_PAD2 = E_PAD + 2 * K_EDGES
ROWS_PER_TILE = NCHUNK * ROWS
SLICE = N_PAD // NS
HR = SLICE // 128


def _dense_body(x_ref, W_ref, as_ref, ad_ref, h_ref, asrc_ref, adst_ref):
    h = jnp.dot(x_ref[...], W_ref[...], preferred_element_type=jnp.float32)
    h_ref[...] = h
    asrc_ref[...] = jnp.dot(h, as_ref[...], preferred_element_type=jnp.float32)
    adst_ref[...] = jnp.dot(h, ad_ref[...], preferred_element_type=jnp.float32)


def _head_body(num_ref, den_ref, bias_ref, W2_ref, b2_ref, y_ref):
    num = jnp.concatenate(
        [num_ref[0:N_PAD, :], num_ref[N_PAD:2 * N_PAD, :]], axis=1)
    den = den_ref[0:N_PAD, 0:1]
    out = num / (den + 1e-16) + bias_ref[...]
    out = jnp.maximum(out, 0.0)
    y = jnp.dot(out, W2_ref[...], preferred_element_type=jnp.float32) + b2_ref[...]
    y_ref[...] = jax.nn.sigmoid(y)


def _sc_edge_body(src_hbm, dst_hbm, asrc_hbm, adst_hbm, h2_hbm,
                  num_out, den_out,
                  src_v0, src_v1, dst_v0, dst_v1, dstsc_v, dstd_v,
                  as_e0, as_e1, ad_e0, ad_e1,
                  ident2_v, w_v,
                  rows_v0, rows_v1, wrows_v,
                  h_loc, asrc_sh, adst_sh, num_acc, den_acc,
                  sem_g0, sem_g1, sem_h0, sem_h1, sem_s, sem_t):
    c = lax.axis_index("c")
    s = lax.axis_index("s")
    src_v = (src_v0, src_v1)
    dst_v = (dst_v0, dst_v1)
    rows_v = (rows_v0, rows_v1)
    as_e = (as_e0, as_e1)
    ad_e = (ad_e0, ad_e1)
    sem_g = (sem_g0, sem_g1)
    sem_h = (sem_h0, sem_h1)

    def _scatter_descs(b):
        ds_ = []
        for gr in range(ROWS):
            ds_.append(pltpu.make_async_copy(
                rows_v[b].at[pl.ds(gr * 128, 128)],
                num_acc.at[dstsc_v.at[gr]], sem_s))
            ds_.append(pltpu.make_async_copy(
                wrows_v.at[pl.ds(gr * 128, 128)],
                den_acc.at[dstd_v.at[gr]], sem_s))
        return ds_

    def _stage(g, b, sync):
        rb = s * ROWS_PER_TILE + g * ROWS
        if sync:
            pltpu.sync_copy(src_hbm.at[pl.ds(rb, ROWS)], src_v[b])
            pltpu.sync_copy(dst_hbm.at[pl.ds(rb, ROWS)], dst_v[b])
        else:
            pltpu.async_copy(src_hbm.at[pl.ds(rb, ROWS)], src_v[b], sem_t)
            pltpu.async_copy(dst_hbm.at[pl.ds(rb, ROWS)], dst_v[b], sem_t)

    def _stage_drain():
        pltpu.make_async_copy(src_hbm.at[pl.ds(0, ROWS)], src_v0, sem_t).wait()
        pltpu.make_async_copy(dst_hbm.at[pl.ds(0, ROWS)], dst_v0, sem_t).wait()

    def _fire_gathers(b):
        for gr in range(ROWS):
            pltpu.async_copy(asrc_sh.at[src_v[b].at[gr]],
                             as_e[b].at[pl.ds(gr * 128, 128)], sem_g[b])
            pltpu.async_copy(adst_sh.at[dst_v[b].at[gr]],
                             ad_e[b].at[pl.ds(gr * 128, 128)], sem_g[b])
            pltpu.async_copy(h_loc.at[src_v[b].at[gr]],
                             rows_v[b].at[pl.ds(gr * 128, 128)], sem_h[b])

    def _gather_descs(b):
        ds_ = []
        for gr in range(ROWS):
            ds_.append(pltpu.make_async_copy(
                asrc_sh.at[src_v[b].at[gr]],
                as_e[b].at[pl.ds(gr * 128, 128)], sem_g[b]))
            ds_.append(pltpu.make_async_copy(
                adst_sh.at[dst_v[b].at[gr]],
                ad_e[b].at[pl.ds(gr * 128, 128)], sem_g[b]))
        return ds_

    def _h_descs(b):
        return [pltpu.make_async_copy(
                    h_loc.at[src_v[b].at[gr]],
                    rows_v[b].at[pl.ds(gr * 128, 128)], sem_h[b])
                for gr in range(ROWS)]

    for gr in range(HR):
        for t in range(8):
            ident2_v[gr, pl.ds(16 * t, 16)] = (
                (jnp.full((16,), s * SLICE + gr * 128 + t * 16, jnp.int32)
                 + lax.iota(jnp.int32, 16)) * 2 + c)

    pltpu.sync_copy(asrc_hbm.at[pl.ds(s * SLICE, SLICE)],
                    asrc_sh.at[pl.ds(s * SLICE, SLICE)])
    pltpu.sync_copy(adst_hbm.at[pl.ds(s * SLICE, SLICE)],
                    adst_sh.at[pl.ds(s * SLICE, SLICE)])
    for gr in range(HR):
        pltpu.async_copy(h2_hbm.at[ident2_v.at[gr]],
                         rows_v0.at[pl.ds(0, 128)], sem_g0).wait()
        pltpu.sync_copy(rows_v0.at[pl.ds(0, 128)],
                        h_loc.at[pl.ds(s * SLICE + gr * 128, 128)])

    def _zero_body(r, carry):
        for d in range(4):
            rows_v0[r, pl.ds(16 * d, 16)] = jnp.zeros((16,), jnp.float32)
            rows_v1[r, pl.ds(16 * d, 16)] = jnp.zeros((16,), jnp.float32)
        return carry
    lax.fori_loop(0, K_EDGES, _zero_body, 0)

    def _zden_body(r, carry):
        wrows_v[r, :] = jnp.zeros((16,), jnp.float32)
        return carry
    lax.fori_loop(0, K_EDGES, _zden_body, 0)
    for gr in range(ROWS):
        for t in range(8):
            dstsc_v[gr, pl.ds(16 * t, 16)] = jnp.zeros((16,), jnp.int32)
            dstd_v[gr, pl.ds(16 * t, 16)] = jnp.zeros((16,), jnp.int32)

    base = s * SLICE
    pltpu.sync_copy(rows_v0, num_acc.at[pl.ds(base, K_EDGES)])
    pltpu.sync_copy(rows_v1, num_acc.at[pl.ds(base + K_EDGES, K_EDGES)])
    pltpu.sync_copy(rows_v0.at[pl.ds(0, SLICE - 2 * K_EDGES)],
                    num_acc.at[pl.ds(base + 2 * K_EDGES,
                                     SLICE - 2 * K_EDGES)])
    pltpu.sync_copy(wrows_v, den_acc.at[pl.ds(s * DEN_SL, K_EDGES)])
    pltpu.sync_copy(wrows_v.at[pl.ds(0, DEN_SL - K_EDGES)],
                    den_acc.at[pl.ds(s * DEN_SL + K_EDGES,
                                     DEN_SL - K_EDGES)])
    plsc.subcore_barrier()

    for gr in range(ROWS):
        pltpu.async_copy(rows_v1.at[pl.ds(gr * 128, 128)],
                         num_acc.at[dstsc_v.at[gr]], sem_s, add=True)
        pltpu.async_copy(wrows_v.at[pl.ds(gr * 128, 128)],
                         den_acc.at[dstd_v.at[gr]], sem_s, add=True)
    _stage(0, 0, sync=True)
    _stage(1, 1, sync=False)
    _fire_gathers(0)

    def _chunk(g, b):
        for d_ in _gather_descs(b):
            d_.wait()
        for d_ in _scatter_descs(1 - b):
            d_.wait()
        _stage_drain()
        _fire_gathers(1 - b)

        def _w_body(gg, carry2):
            for t in range(8):
                sl = pl.ds(16 * t, 16)
                sle = pl.ds(gg * 128 + t * 16, 16)
                di = dst_v[b][gg, sl]
                e = as_e[b][sle] + ad_e[b][sle]
                e = jnp.where(e < 0, e * jnp.float32(0.2), e)
                w_v[sle] = jnp.exp(e)
                dl = di - c * DEN_HALF
                ok = (dl >= 0) & (dl < DEN_HALF)
                trash = DEN_HALF + (di & 63)
                dstd_v[gg, sl] = jnp.where(ok, dl, trash)
            return carry2
        lax.fori_loop(0, ROWS, _w_body, 0)

        for d_ in _h_descs(b):
            d_.wait()

        def _scale_blk(bk, carry2):
            r0 = bk * 16
            w16 = w_v[pl.ds(r0, 16)]
            for l in range(16):
                wb = jnp.broadcast_to(w16[l], (16,))
                for d in range(4):
                    sl = pl.ds(16 * d, 16)
                    rows_v[b][r0 + l, sl] = rows_v[b][r0 + l, sl] * wb
                wrows_v[r0 + l, :] = wb
            return carry2
        lax.fori_loop(0, K_EDGES // 16, _scale_blk, 0)

        for gr in range(ROWS):
            for t in range(8):
                sl = pl.ds(16 * t, 16)
                dstsc_v[gr, sl] = dst_v[b][gr, sl]
        for gr in range(ROWS):
            pltpu.async_copy(rows_v[b].at[pl.ds(gr * 128, 128)],
                             num_acc.at[dstsc_v.at[gr]], sem_s, add=True)
            pltpu.async_copy(wrows_v.at[pl.ds(gr * 128, 128)],
                             den_acc.at[dstd_v.at[gr]], sem_s, add=True)

        _stage(g + 2, b, sync=False)

    def _outer(g2, carry):
        _chunk(g2 * 2, 0)
        _chunk(g2 * 2 + 1, 1)
        return carry
    lax.fori_loop(0, NCHUNK // 2, _outer, 0)

    for d_ in _gather_descs(0):
        d_.wait()
    for d_ in _h_descs(0):
        d_.wait()
    _stage_drain()
    for d_ in _scatter_descs(1):
        d_.wait()

    plsc.subcore_barrier()
    ob = c * N_PAD + s * SLICE
    pltpu.sync_copy(num_acc.at[pl.ds(s * SLICE, SLICE)],
                    num_out.at[pl.ds(ob, SLICE)])
    pltpu.sync_copy(den_acc.at[pl.ds(s * DEN_SL, DEN_SL)],
                    den_out.at[pl.ds(c * DEN_RR + s * DEN_SL, DEN_SL)])


_sc_edge = functools.partial(
    pl.kernel,
    out_type=[
        jax.ShapeDtypeStruct((2 * N_PAD, D_HALF), jnp.float32),
        jax.ShapeDtypeStruct((2 * DEN_RR, 16), jnp.float32),
    ],
    mesh=plsc.VectorSubcoreMesh(core_axis_name="c", subcore_axis_name="s"),
    compiler_params=pltpu.CompilerParams(needs_layout_passes=False,
                                         use_tc_tiling_on_sc=False),
    scratch_types=[
        pltpu.VMEM((ROWS, 128), jnp.int32),
        pltpu.VMEM((ROWS, 128), jnp.int32),
        pltpu.VMEM((ROWS, 128), jnp.int32),
        pltpu.VMEM((ROWS, 128), jnp.int32),
        pltpu.VMEM((ROWS, 128), jnp.int32),
        pltpu.VMEM((ROWS, 128), jnp.int32),
        pltpu.VMEM((K_EDGES,), jnp.float32),
        pltpu.VMEM((K_EDGES,), jnp.float32),
        pltpu.VMEM((K_EDGES,), jnp.float32),
        pltpu.VMEM((K_EDGES,), jnp.float32),
        pltpu.VMEM((HR, 128), jnp.int32),
        pltpu.VMEM((K_EDGES,), jnp.float32),
        pltpu.VMEM((K_EDGES, D_HALF), jnp.float32),
        pltpu.VMEM((K_EDGES, D_HALF), jnp.float32),
        pltpu.VMEM((K_EDGES, 16), jnp.float32),
        pltpu.VMEM_SHARED((N_PAD, D_HALF), jnp.float32),
        pltpu.VMEM_SHARED((N_PAD,), jnp.float32),
        pltpu.VMEM_SHARED((N_PAD,), jnp.float32),
        pltpu.VMEM_SHARED((N_PAD, D_HALF), jnp.float32),
        pltpu.VMEM_SHARED((DEN_RR, 16), jnp.float32),
        pltpu.SemaphoreType.DMA,
        pltpu.SemaphoreType.DMA,
        pltpu.SemaphoreType.DMA,
        pltpu.SemaphoreType.DMA,
        pltpu.SemaphoreType.DMA,
        pltpu.SemaphoreType.DMA,
    ],
)(_sc_edge_body)


def kernel(x, edge_index, W, att_src, att_dst, bias, W2, b2):
    h, a_src, a_dst = pl.pallas_call(
        _dense_body,
        out_shape=[
            jax.ShapeDtypeStruct((N_NODES, D_HID), jnp.float32),
            jax.ShapeDtypeStruct((N_NODES, 1), jnp.float32),
            jax.ShapeDtypeStruct((N_NODES, 1), jnp.float32),
        ],
    )(x, W, att_src.reshape(D_HID, 1), att_dst.reshape(D_HID, 1))

    loop = jnp.arange(N_NODES, dtype=jnp.int32)
    src = jnp.concatenate([edge_index[0].astype(jnp.int32), loop,
                           jnp.zeros((E_PAD2 - E_TOT,), jnp.int32)])
    dst = jnp.concatenate([edge_index[1].astype(jnp.int32), loop,
                           jnp.full((E_PAD2 - E_TOT,), N_NODES, jnp.int32)])
    src2 = src.reshape(E_PAD2 // 128, 128)
    dst2 = dst.reshape(E_PAD2 // 128, 128)
    asrc_p = jnp.pad(a_src[:, 0], (0, N_PAD - N_NODES))
    adst_p = jnp.pad(a_dst[:, 0], (0, N_PAD - N_NODES))
    hp = jnp.pad(h, ((0, N_PAD - N_NODES), (0, 0)))
    h2 = hp.reshape(2 * N_PAD, D_HALF)

    num_flat, den_flat = _sc_edge(src2, dst2, asrc_p, adst_p, h2)
    den = jnp.concatenate([den_flat[:DEN_HALF, 0],
                           den_flat[DEN_RR:DEN_RR + DEN_HALF, 0]])
    den = den.reshape(N_PAD, 1)

    y = pl.pallas_call(
        _head_body,
        out_shape=jax.ShapeDtypeStruct((N_PAD, 1), jnp.float32),
    )(num_flat, den, bias.reshape(1, D_HID), W2, b2.reshape(1, 1))
    return y[:N_NODES]
